# baseline (speedup 1.0000x reference)
import jax
import jax.numpy as jnp
from jax import lax
from jax.experimental import pallas as pl
from jax.experimental.pallas import tpu as pltpu
from jax.experimental.pallas import tpu_sc as plsc

B = 16384
D = 64

_info = plsc.get_sparse_core_info()
NC = _info.num_cores
NS = _info.num_subcores
L = _info.num_lanes
NW = NC * NS
BPW = B // NW
CHUNK = 128
NCHUNK = BPW // CHUNK


def _bpr_body(user_r, itemi_r, itemj_r, eu, ei, out_i, out_j,
              idx_u, idx_i, idx_j, u_rows, vi_rows, vj_rows,
              oi_v, oj_v, sem):
    wid = lax.axis_index("s") * NC + lax.axis_index("c")
    base = wid * BPW

    pltpu.sync_copy(user_r.at[wid], idx_u)
    pltpu.sync_copy(itemi_r.at[wid], idx_i)
    pltpu.sync_copy(itemj_r.at[wid], idx_j)

    copies = []
    for c in range(NCHUNK):
        sl = pl.ds(c * CHUNK, CHUNK)
        copies.append(pltpu.async_copy(eu.at[idx_u.at[c]], u_rows.at[sl], sem))
        copies.append(pltpu.async_copy(ei.at[idx_i.at[c]], vi_rows.at[sl], sem))
        copies.append(pltpu.async_copy(ei.at[idx_j.at[c]], vj_rows.at[sl], sem))
    for cp in copies:
        cp.wait()

    def g_body(g, carry):
        row = g * L + lax.iota(jnp.int32, L)
        acc_i = jnp.zeros((L,), jnp.float32)
        acc_j = jnp.zeros((L,), jnp.float32)
        for d in range(D):
            col = jnp.full((L,), d, jnp.int32)
            uv = plsc.load_gather(u_rows, [row, col])
            iv = plsc.load_gather(vi_rows, [row, col])
            jv = plsc.load_gather(vj_rows, [row, col])
            acc_i = acc_i + uv * iv
            acc_j = acc_j + uv * jv
        oi_v[pl.ds(g * L, L)] = acc_i
        oj_v[pl.ds(g * L, L)] = acc_j
        return carry

    lax.fori_loop(0, BPW // L, g_body, 0)

    pltpu.sync_copy(oi_v, out_i.at[pl.ds(base, BPW)])
    pltpu.sync_copy(oj_v, out_j.at[pl.ds(base, BPW)])


def kernel(user, item_i, item_j, embed_user, embed_item):
    user_r = user.reshape(NW, NCHUNK, CHUNK)
    itemi_r = item_i.reshape(NW, NCHUNK, CHUNK)
    itemj_r = item_j.reshape(NW, NCHUNK, CHUNK)
    mesh = plsc.VectorSubcoreMesh(core_axis_name="c", subcore_axis_name="s")
    f = pl.kernel(
        _bpr_body,
        mesh=mesh,
        out_type=(jax.ShapeDtypeStruct((B,), jnp.float32),
                  jax.ShapeDtypeStruct((B,), jnp.float32)),
        scratch_types=[
            pltpu.VMEM((NCHUNK, CHUNK), jnp.int32),
            pltpu.VMEM((NCHUNK, CHUNK), jnp.int32),
            pltpu.VMEM((NCHUNK, CHUNK), jnp.int32),
            pltpu.VMEM((BPW, D), jnp.float32),
            pltpu.VMEM((BPW, D), jnp.float32),
            pltpu.VMEM((BPW, D), jnp.float32),
            pltpu.VMEM((BPW,), jnp.float32),
            pltpu.VMEM((BPW,), jnp.float32),
            pltpu.SemaphoreType.DMA,
        ],
        compiler_params=pltpu.CompilerParams(needs_layout_passes=False,
                                             use_tc_tiling_on_sc=False),
    )
    return f(user_r, itemi_r, itemj_r, embed_user, embed_item)
